# Initial kernel scaffold; baseline (speedup 1.0000x reference)
import jax
import jax.numpy as jnp
from jax.experimental import pallas as pl
from jax.experimental.pallas import tpu as pltpu

NUM_EMBEDDINGS = 512
EMBEDDING_DIM = 64
COMMITMENT_COST = 0.25
N_ROWS = 16 * 32 * 32
BLOCK_ROWS = 2048
GRID = N_ROWS // BLOCK_ROWS


def _vq_body(x_ref, e_ref, enc_ref, qst_ref, loss_ref, perp_ref,
             sse_acc, cnt_acc):
    i = pl.program_id(0)
    x = x_ref[...]
    e = e_ref[...]

    sx = jnp.sum(x * x, axis=1, keepdims=True)
    se = jnp.sum(e * e, axis=1)
    mm = jax.lax.dot_general(x, e, (((1,), (1,)), ((), ())))
    d = sx + se - 2.0 * mm

    m = jnp.min(d, axis=1, keepdims=True)
    iota = jax.lax.broadcasted_iota(jnp.int32, d.shape, 1)
    idx = jnp.min(jnp.where(d == m, iota, NUM_EMBEDDINGS), axis=1,
                  keepdims=True)
    enc = (iota == idx).astype(jnp.float32)
    enc_ref[...] = enc

    quant = jnp.dot(enc, e)
    diff = quant - x
    qst_ref[...] = x + diff

    sse_p = jnp.sum(diff * diff)
    cnt_p = jnp.sum(enc, axis=0, keepdims=True)

    @pl.when(i == 0)
    def _init():
        sse_acc[0, 0] = 0.0
        cnt_acc[...] = jnp.zeros_like(cnt_acc)

    sse_acc[0, 0] += sse_p
    cnt_acc[...] += cnt_p

    @pl.when(i == GRID - 1)
    def _fini():
        mean = sse_acc[0, 0] / float(N_ROWS * EMBEDDING_DIM)
        loss_ref[0, 0] = mean + COMMITMENT_COST * mean
        p = cnt_acc[...] / float(N_ROWS)
        perp_ref[0, 0] = jnp.exp(-jnp.sum(p * jnp.log(p + 1e-10)))


def kernel(inputs, labels, embedding):
    del labels
    x = jnp.transpose(inputs, (0, 2, 3, 1))
    input_shape = x.shape
    flat = x.reshape(-1, EMBEDDING_DIM)

    enc, qst, loss, perp = pl.pallas_call(
        _vq_body,
        grid=(GRID,),
        in_specs=[
            pl.BlockSpec((BLOCK_ROWS, EMBEDDING_DIM), lambda i: (i, 0)),
            pl.BlockSpec((NUM_EMBEDDINGS, EMBEDDING_DIM), lambda i: (0, 0)),
        ],
        out_specs=[
            pl.BlockSpec((BLOCK_ROWS, NUM_EMBEDDINGS), lambda i: (i, 0)),
            pl.BlockSpec((BLOCK_ROWS, EMBEDDING_DIM), lambda i: (i, 0)),
            pl.BlockSpec((1, 1), lambda i: (0, 0)),
            pl.BlockSpec((1, 1), lambda i: (0, 0)),
        ],
        out_shape=[
            jax.ShapeDtypeStruct((N_ROWS, NUM_EMBEDDINGS), jnp.float32),
            jax.ShapeDtypeStruct((N_ROWS, EMBEDDING_DIM), jnp.float32),
            jax.ShapeDtypeStruct((1, 1), jnp.float32),
            jax.ShapeDtypeStruct((1, 1), jnp.float32),
        ],
        scratch_shapes=[
            pltpu.SMEM((1, 1), jnp.float32),
            pltpu.VMEM((1, NUM_EMBEDDINGS), jnp.float32),
        ],
    )(flat, embedding)

    quantized_st = qst.reshape(input_shape)
    return (loss[0, 0], jnp.transpose(quantized_st, (0, 3, 1, 2)),
            perp[0, 0], enc)


# speedup vs baseline: 1.9759x; 1.9759x over previous
import jax
import jax.numpy as jnp
from jax.experimental import pallas as pl
from jax.experimental.pallas import tpu as pltpu

NUM_EMBEDDINGS = 512
EMBEDDING_DIM = 64
COMMITMENT_COST = 0.25
N_ROWS = 16 * 32 * 32
BLOCK_ROWS = 2048
GRID = N_ROWS // BLOCK_ROWS


def _vq_body(x_ref, e_ref, enc_ref, qst_ref, loss_ref, perp_ref,
             sse_acc, cnt_acc):
    i = pl.program_id(0)
    x = x_ref[...]
    e = e_ref[...]

    sx = jnp.sum(x * x, axis=1, keepdims=True)
    se = jnp.sum(e * e, axis=1)
    mm = jax.lax.dot_general(x, e, (((1,), (1,)), ((), ())))
    d = sx + se - 2.0 * mm

    m = jnp.min(d, axis=1, keepdims=True)
    iota = jax.lax.broadcasted_iota(jnp.int32, d.shape, 1)
    idx = jnp.min(jnp.where(d == m, iota, NUM_EMBEDDINGS), axis=1,
                  keepdims=True)
    enc = (iota == idx).astype(jnp.float32)
    enc_ref[...] = enc

    quant = jnp.dot(enc, e)
    diff = quant - x
    qst_ref[...] = x + diff

    sse_p = jnp.sum(diff * diff)
    cnt_p = jnp.sum(enc, axis=0, keepdims=True)

    @pl.when(i == 0)
    def _init():
        sse_acc[0, 0] = 0.0
        cnt_acc[...] = jnp.zeros_like(cnt_acc)

    sse_acc[0, 0] += sse_p
    cnt_acc[...] += cnt_p

    @pl.when(i == GRID - 1)
    def _fini():
        mean = sse_acc[0, 0] / float(N_ROWS * EMBEDDING_DIM)
        loss_ref[...] = jnp.full((1, 1), mean + COMMITMENT_COST * mean,
                                 dtype=jnp.float32)
        p = cnt_acc[...] / float(N_ROWS)
        ent = -jnp.sum(p * jnp.log(p + 1e-10))
        perp_ref[...] = jnp.exp(jnp.full((1, 1), ent, dtype=jnp.float32))


def kernel(inputs, labels, embedding):
    del labels
    x = jnp.transpose(inputs, (0, 2, 3, 1))
    input_shape = x.shape
    flat = x.reshape(-1, EMBEDDING_DIM)

    enc, qst, loss, perp = pl.pallas_call(
        _vq_body,
        grid=(GRID,),
        in_specs=[
            pl.BlockSpec((BLOCK_ROWS, EMBEDDING_DIM), lambda i: (i, 0)),
            pl.BlockSpec((NUM_EMBEDDINGS, EMBEDDING_DIM), lambda i: (0, 0)),
        ],
        out_specs=[
            pl.BlockSpec((BLOCK_ROWS, NUM_EMBEDDINGS), lambda i: (i, 0)),
            pl.BlockSpec((BLOCK_ROWS, EMBEDDING_DIM), lambda i: (i, 0)),
            pl.BlockSpec((1, 1), lambda i: (0, 0)),
            pl.BlockSpec((1, 1), lambda i: (0, 0)),
        ],
        out_shape=[
            jax.ShapeDtypeStruct((N_ROWS, NUM_EMBEDDINGS), jnp.float32),
            jax.ShapeDtypeStruct((N_ROWS, EMBEDDING_DIM), jnp.float32),
            jax.ShapeDtypeStruct((1, 1), jnp.float32),
            jax.ShapeDtypeStruct((1, 1), jnp.float32),
        ],
        scratch_shapes=[
            pltpu.SMEM((1, 1), jnp.float32),
            pltpu.VMEM((1, NUM_EMBEDDINGS), jnp.float32),
        ],
    )(flat, embedding)

    quantized_st = qst.reshape(input_shape)
    return (loss[0, 0], jnp.transpose(quantized_st, (0, 3, 1, 2)),
            perp[0, 0], enc)
